# Initial kernel scaffold; baseline (speedup 1.0000x reference)
import jax
import jax.numpy as jnp
from jax.experimental import pallas as pl


def kernel(x0, x1, noise0, noise1, W_pre0, b_pre0, W_pre1, b_pre1, W_router, enc_w1, enc_b1, enc_w2, enc_b2, enc_w3, enc_b3, enc_w4, enc_b4, dec0_w1, dec0_b1, dec0_w2, dec0_b2, dec0_w3, dec0_b3, dec0_w4, dec0_b4, dec1_w1, dec1_b1, dec1_w2, dec1_b2, dec1_w3, dec1_b3, dec1_w4, dec1_b4):
    raise NotImplementedError("write your pallas kernel here")


# speedup vs baseline: 1.1915x; 1.1915x over previous
import numpy as np
import jax
import jax.numpy as jnp
from jax.experimental import pallas as pl
from jax.experimental.pallas import tpu as pltpu

B = 1024
E = 8
K = 2
F = 512
C = 128
N_SEL = 920
N_TOT = 2 * N_SEL

_INTERPRET = False


def _dotT(a, w):
    return jax.lax.dot_general(a, w, (((1,), (1,)), ((), ())),
                               preferred_element_type=jnp.float32)


def _lrelu(x):
    return jnp.where(x >= 0, x, 0.01 * x)


def _pre_body(x_ref, w_ref, b_ref, o_ref):
    o_ref[...] = _dotT(x_ref[...], w_ref[...]) + b_ref[...]


def _pre(x, w, b):
    return pl.pallas_call(
        _pre_body,
        out_shape=jax.ShapeDtypeStruct((B, F), jnp.float32),
        interpret=_INTERPRET,
    )(x, w, b.reshape(1, F))


def _route_body(m_ref, noise_ref, wr_ref, ei_ref, ohg1_ref, ohg2_ref, bal_ref):
    m = m_ref[...]
    sel = _dotT(m, wr_ref[...]) + noise_ref[...]
    eidx = jax.lax.broadcasted_iota(jnp.int32, (B, E), 1)
    g1 = jnp.max(sel, axis=1, keepdims=True)
    i1 = jnp.argmax(sel, axis=1)[:, None]
    oh1 = (eidx == i1).astype(jnp.float32)
    sel2 = jnp.where(oh1 > 0, -jnp.inf, sel)
    g2 = jnp.max(sel2, axis=1, keepdims=True)
    i2 = jnp.argmax(sel2, axis=1)[:, None]
    oh2 = (eidx == i2).astype(jnp.float32)
    ohg1_ref[...] = oh1 * g1
    ohg2_ref[...] = oh2 * g2
    d1 = oh1 * (g1 != 0).astype(jnp.float32)
    d2 = oh2 * (g2 != 0).astype(jnp.float32)
    ohd = jnp.concatenate([d1, d2], axis=1)
    ei_ref[...] = jax.lax.dot_general(ohd, m, (((0,), (0,)), ((), ())),
                                      preferred_element_type=jnp.float32)
    colsum = jnp.sum(0.5 * (oh1 + oh2), axis=0, keepdims=True)
    proxy = jnp.mean(sel, axis=0, keepdims=True)
    bal_ref[...] = (jnp.sum(proxy * colsum) * (E * E) / (B * E)).reshape(1, 1)


def _route(m, noise, wr):
    return pl.pallas_call(
        _route_body,
        out_shape=(
            jax.ShapeDtypeStruct((2 * E, F), jnp.float32),
            jax.ShapeDtypeStruct((B, E), jnp.float32),
            jax.ShapeDtypeStruct((B, E), jnp.float32),
            jax.ShapeDtypeStruct((1, 1), jnp.float32),
        ),
        interpret=_INTERPRET,
    )(m, noise, wr)


def _enc_body(ei_ref, w1_ref, b1_ref, w2_ref, b2_ref, w3_ref, b3_ref,
              w4_ref, b4_ref, eo_ref):
    x = ei_ref[0]
    h = jnp.maximum(_dotT(x, w1_ref[0]) + b1_ref[0], 0.0)
    h = jnp.maximum(_dotT(h, w2_ref[0]) + b2_ref[0], 0.0)
    h = jnp.maximum(_dotT(h, w3_ref[0]) + b3_ref[0], 0.0)
    eo_ref[0] = _dotT(h, w4_ref[0]) + b4_ref[0]


def _encoder(ei, w1, b1, w2, b2, w3, b3, w4, b4):
    n4 = 2 * K
    spec_w = lambda s: pl.BlockSpec((1,) + s, lambda e: (e, 0, 0))
    return pl.pallas_call(
        _enc_body,
        grid=(E,),
        in_specs=[
            pl.BlockSpec((1, n4, F), lambda e: (e, 0, 0)),
            spec_w((500, F)), spec_w((1, 500)),
            spec_w((500, 500)), spec_w((1, 500)),
            spec_w((2000, 500)), spec_w((1, 2000)),
            spec_w((C, 2000)), spec_w((1, C)),
        ],
        out_specs=pl.BlockSpec((1, n4, C), lambda e: (e, 0, 0)),
        out_shape=jax.ShapeDtypeStruct((E, n4, C), jnp.float32),
        interpret=_INTERPRET,
    )(ei, w1, b1.reshape(E, 1, 500), w2, b2.reshape(E, 1, 500),
      w3, b3.reshape(E, 1, 2000), w4, b4.reshape(E, 1, C))


def _combine_body(eo0_ref, eo1_ref, ohg1_ref, ohg2_ref,
                  mid0_ref, mid1_ref, out_ref):
    m0 = jnp.dot(ohg1_ref[...], eo0_ref[...],
                 preferred_element_type=jnp.float32)
    m1 = jnp.dot(ohg2_ref[...], eo1_ref[...],
                 preferred_element_type=jnp.float32)
    mid0_ref[...] = m0
    mid1_ref[...] = m1
    out_ref[...] = m0 + m1


def _combine(eo0, eo1, ohg1, ohg2):
    return pl.pallas_call(
        _combine_body,
        out_shape=(
            jax.ShapeDtypeStruct((B, C), jnp.float32),
            jax.ShapeDtypeStruct((B, C), jnp.float32),
            jax.ShapeDtypeStruct((B, C), jnp.float32),
        ),
        interpret=_INTERPRET,
    )(eo0, eo1, ohg1, ohg2)


def _mmd_masks(seed):
    rng = np.random.default_rng(seed)
    i1 = rng.permutation(B)[:N_SEL]
    i2 = rng.permutation(B)[:N_SEL]
    w0 = np.zeros((B,), np.float32)
    w0[i1] = 1.0
    w1 = np.zeros((B,), np.float32)
    w1[i2] = 1.0
    return w0, w1


_MMD_R = 512


def _mmd_body(T_ref, Trow_ref, srow_ref, scol_ref, dl_ref):
    v = pl.program_id(0)
    t = pl.program_id(1)
    T = T_ref[0]
    s = srow_ref[0]
    m = jnp.abs(s)
    sq = jnp.sum(T * T, axis=1, keepdims=True)
    S1 = jnp.sum(jnp.dot(m, sq, preferred_element_type=jnp.float32))
    sumvec = jnp.dot(m, T, preferred_element_type=jnp.float32)
    sum_l2 = 2.0 * N_TOT * S1 - 2.0 * jnp.sum(sumvec * sumvec)
    bw = sum_l2 / (N_TOT * N_TOT - N_TOT) / 4.0
    Trow = Trow_ref[0]
    sq_r = jnp.sum(Trow * Trow, axis=1, keepdims=True)
    s_r = scol_ref[0]
    G = jax.lax.dot_general(Trow, T, (((1,), (1,)), ((), ())),
                            preferred_element_type=jnp.float32)
    L2 = sq_r + jnp.transpose(sq) - 2.0 * G
    kern = (jnp.exp(-L2 / bw) + jnp.exp(-L2 / (2.0 * bw))
            + jnp.exp(-L2 / (4.0 * bw)) + jnp.exp(-L2 / (8.0 * bw))
            + jnp.exp(-L2 / (16.0 * bw)))
    acc = jnp.sum(kern * (s_r * s))

    @pl.when(jnp.logical_and(v == 0, t == 0))
    def _():
        dl_ref[...] = jnp.zeros((1, 1), jnp.float32)

    dl_ref[...] = dl_ref[...] + (-acc / (N_SEL * N_SEL)).reshape(1, 1)


def _mmd(Ts, srows, scols):
    return pl.pallas_call(
        _mmd_body,
        grid=(2, 2 * B // _MMD_R),
        in_specs=[
            pl.BlockSpec((1, 2 * B, C), lambda v, t: (v, 0, 0)),
            pl.BlockSpec((1, _MMD_R, C), lambda v, t: (v, t, 0)),
            pl.BlockSpec((1, 1, 2 * B), lambda v, t: (v, 0, 0)),
            pl.BlockSpec((1, _MMD_R, 1), lambda v, t: (v, t, 0)),
        ],
        out_specs=pl.BlockSpec((1, 1), lambda v, t: (0, 0)),
        out_shape=jax.ShapeDtypeStruct((1, 1), jnp.float32),
        interpret=_INTERPRET,
    )(Ts, Ts, srows, scols)


def _dec_body(f_ref, w1_ref, b1_ref, w2_ref, b2_ref, w3_ref, b3_ref,
              w4_ref, b4_ref, o_ref):
    h = _lrelu(_dotT(f_ref[...], w1_ref[...]) + b1_ref[...])
    h = _lrelu(_dotT(h, w2_ref[...]) + b2_ref[...])
    h = _lrelu(_dotT(h, w3_ref[...]) + b3_ref[...])
    o_ref[...] = _dotT(h, w4_ref[...]) + b4_ref[...]


def _decoder(fused, w1, b1, w2, b2, w3, b3, w4, b4):
    din = w4.shape[0]
    return pl.pallas_call(
        _dec_body,
        out_shape=jax.ShapeDtypeStruct((B, din), jnp.float32),
        interpret=_INTERPRET,
    )(fused, w1, b1.reshape(1, -1), w2, b2.reshape(1, -1),
      w3, b3.reshape(1, -1), w4, b4.reshape(1, -1))


def kernel(x0, x1, noise0, noise1, W_pre0, b_pre0, W_pre1, b_pre1, W_router,
           enc_w1, enc_b1, enc_w2, enc_b2, enc_w3, enc_b3, enc_w4, enc_b4,
           dec0_w1, dec0_b1, dec0_w2, dec0_b2, dec0_w3, dec0_b3, dec0_w4,
           dec0_b4, dec1_w1, dec1_b1, dec1_w2, dec1_b2, dec1_w3, dec1_b3,
           dec1_w4, dec1_b4):
    m0 = _pre(x0, W_pre0, b_pre0)
    m1 = _pre(x1, W_pre1, b_pre1)

    ei0, ohg1_0, ohg2_0, bal0 = _route(m0, noise0, W_router)
    ei1, ohg1_1, ohg2_1, bal1 = _route(m1, noise1, W_router)

    ei = jnp.stack([ei0.reshape(K, E, F), ei1.reshape(K, E, F)], axis=0)
    ei = jnp.transpose(ei, (2, 0, 1, 3)).reshape(E, 2 * K, F)

    eo = _encoder(ei, enc_w1, enc_b1, enc_w2, enc_b2, enc_w3, enc_b3,
                  enc_w4, enc_b4)

    mid0_0, mid1_0, o0 = _combine(eo[:, 0, :], eo[:, 1, :], ohg1_0, ohg2_0)
    mid0_1, mid1_1, o1 = _combine(eo[:, 2, :], eo[:, 3, :], ohg1_1, ohg2_1)

    srows = []
    for seed in (0, 1):
        w0, w1 = _mmd_masks(seed)
        srows.append(np.concatenate([w0, -w1]).reshape(1, 2 * B))
    srows_np = np.stack(srows)
    srows = jnp.asarray(srows_np)
    scols = jnp.asarray(np.transpose(srows_np, (0, 2, 1)))

    Ts = jnp.stack([jnp.concatenate([mid0_0, mid1_0], axis=0),
                    jnp.concatenate([mid0_1, mid1_1], axis=0)])
    dist = _mmd(Ts, srows, scols)

    fused = jnp.concatenate([o0, o1], axis=1)
    rec0 = _decoder(fused, dec0_w1, dec0_b1, dec0_w2, dec0_b2,
                    dec0_w3, dec0_b3, dec0_w4, dec0_b4)
    rec1 = _decoder(fused, dec1_w1, dec1_b1, dec1_w2, dec1_b2,
                    dec1_w3, dec1_b3, dec1_w4, dec1_b4)

    bal = (bal0 + bal1).reshape(())
    dl = dist.reshape(())
    return fused, rec0, rec1, bal, dl


# speedup vs baseline: 1.2662x; 1.0627x over previous
import numpy as np
import jax
import jax.numpy as jnp
from jax.experimental import pallas as pl
from jax.experimental.pallas import tpu as pltpu

B = 1024
E = 8
K = 2
F = 512
C = 128
N_SEL = 920
N_TOT = 2 * N_SEL

_INTERPRET = False


def _dotT(a, w):
    return jax.lax.dot_general(a, w, (((1,), (1,)), ((), ())),
                               preferred_element_type=jnp.float32)


def _dotT16(a, w):
    return jax.lax.dot_general(a.astype(jnp.bfloat16), w.astype(jnp.bfloat16),
                               (((1,), (1,)), ((), ())),
                               preferred_element_type=jnp.float32)


def _lrelu(x):
    return jnp.where(x >= 0, x, 0.01 * x)


def _pre_body(x_ref, w_ref, b_ref, o_ref):
    o_ref[...] = _dotT16(x_ref[...], w_ref[...]) + b_ref[...]


def _pre(x, w, b):
    return pl.pallas_call(
        _pre_body,
        out_shape=jax.ShapeDtypeStruct((B, F), jnp.float32),
        interpret=_INTERPRET,
    )(x, w, b.reshape(1, F))


def _route_body(m_ref, noise_ref, wr_ref, ei_ref, ohg1_ref, ohg2_ref, bal_ref):
    m = m_ref[...]
    sel = _dotT(m, wr_ref[...]) + noise_ref[...]
    eidx = jax.lax.broadcasted_iota(jnp.int32, (B, E), 1)
    g1 = jnp.max(sel, axis=1, keepdims=True)
    i1 = jnp.argmax(sel, axis=1)[:, None]
    oh1 = (eidx == i1).astype(jnp.float32)
    sel2 = jnp.where(oh1 > 0, -jnp.inf, sel)
    g2 = jnp.max(sel2, axis=1, keepdims=True)
    i2 = jnp.argmax(sel2, axis=1)[:, None]
    oh2 = (eidx == i2).astype(jnp.float32)
    ohg1_ref[...] = oh1 * g1
    ohg2_ref[...] = oh2 * g2
    d1 = oh1 * (g1 != 0).astype(jnp.float32)
    d2 = oh2 * (g2 != 0).astype(jnp.float32)
    ohd = jnp.concatenate([d1, d2], axis=1)
    ei_ref[...] = jax.lax.dot_general(ohd, m, (((0,), (0,)), ((), ())),
                                      preferred_element_type=jnp.float32)
    colsum = jnp.sum(0.5 * (oh1 + oh2), axis=0, keepdims=True)
    proxy = jnp.mean(sel, axis=0, keepdims=True)
    bal_ref[...] = (jnp.sum(proxy * colsum) * (E * E) / (B * E)).reshape(1, 1)


def _route(m, noise, wr):
    return pl.pallas_call(
        _route_body,
        out_shape=(
            jax.ShapeDtypeStruct((2 * E, F), jnp.float32),
            jax.ShapeDtypeStruct((B, E), jnp.float32),
            jax.ShapeDtypeStruct((B, E), jnp.float32),
            jax.ShapeDtypeStruct((1, 1), jnp.float32),
        ),
        interpret=_INTERPRET,
    )(m, noise, wr)


def _enc_body(ei_ref, w1_ref, b1_ref, w2_ref, b2_ref, w3_ref, b3_ref,
              w4_ref, b4_ref, eo_ref):
    x = ei_ref[0]
    h = jnp.maximum(_dotT(x, w1_ref[0]) + b1_ref[0], 0.0)
    h = jnp.maximum(_dotT(h, w2_ref[0]) + b2_ref[0], 0.0)
    h = jnp.maximum(_dotT(h, w3_ref[0]) + b3_ref[0], 0.0)
    eo_ref[0] = _dotT(h, w4_ref[0]) + b4_ref[0]


def _encoder(ei, w1, b1, w2, b2, w3, b3, w4, b4):
    n4 = 2 * K
    spec_w = lambda s: pl.BlockSpec((1,) + s, lambda e: (e, 0, 0))
    return pl.pallas_call(
        _enc_body,
        grid=(E,),
        in_specs=[
            pl.BlockSpec((1, n4, F), lambda e: (e, 0, 0)),
            spec_w((500, F)), spec_w((1, 500)),
            spec_w((500, 500)), spec_w((1, 500)),
            spec_w((2000, 500)), spec_w((1, 2000)),
            spec_w((C, 2000)), spec_w((1, C)),
        ],
        out_specs=pl.BlockSpec((1, n4, C), lambda e: (e, 0, 0)),
        out_shape=jax.ShapeDtypeStruct((E, n4, C), jnp.float32),
        interpret=_INTERPRET,
    )(ei, w1, b1.reshape(E, 1, 500), w2, b2.reshape(E, 1, 500),
      w3, b3.reshape(E, 1, 2000), w4, b4.reshape(E, 1, C))


def _combine_body(eo0_ref, eo1_ref, ohg1_ref, ohg2_ref,
                  mid0_ref, mid1_ref, out_ref):
    m0 = jnp.dot(ohg1_ref[...], eo0_ref[...],
                 preferred_element_type=jnp.float32)
    m1 = jnp.dot(ohg2_ref[...], eo1_ref[...],
                 preferred_element_type=jnp.float32)
    mid0_ref[...] = m0
    mid1_ref[...] = m1
    out_ref[...] = m0 + m1


def _combine(eo0, eo1, ohg1, ohg2):
    return pl.pallas_call(
        _combine_body,
        out_shape=(
            jax.ShapeDtypeStruct((B, C), jnp.float32),
            jax.ShapeDtypeStruct((B, C), jnp.float32),
            jax.ShapeDtypeStruct((B, C), jnp.float32),
        ),
        interpret=_INTERPRET,
    )(eo0, eo1, ohg1, ohg2)


def _mmd_masks(seed):
    rng = np.random.default_rng(seed)
    i1 = rng.permutation(B)[:N_SEL]
    i2 = rng.permutation(B)[:N_SEL]
    w0 = np.zeros((B,), np.float32)
    w0[i1] = 1.0
    w1 = np.zeros((B,), np.float32)
    w1[i2] = 1.0
    return w0, w1


_MMD_R = 512


def _mmd_body(T_ref, Trow_ref, srow_ref, scol_ref, dl_ref):
    v = pl.program_id(0)
    t = pl.program_id(1)
    T = T_ref[0]
    s = srow_ref[0]
    m = jnp.abs(s)
    sq = jnp.sum(T * T, axis=1, keepdims=True)
    S1 = jnp.sum(jnp.dot(m, sq, preferred_element_type=jnp.float32))
    sumvec = jnp.dot(m, T, preferred_element_type=jnp.float32)
    sum_l2 = 2.0 * N_TOT * S1 - 2.0 * jnp.sum(sumvec * sumvec)
    bw = sum_l2 / (N_TOT * N_TOT - N_TOT) / 4.0
    Trow = Trow_ref[0]
    sq_r = jnp.sum(Trow * Trow, axis=1, keepdims=True)
    s_r = scol_ref[0]
    G = jax.lax.dot_general(Trow, T, (((1,), (1,)), ((), ())),
                            preferred_element_type=jnp.float32)
    L2 = sq_r + jnp.transpose(sq) - 2.0 * G
    z = jnp.exp(-L2 / (16.0 * bw))
    z2 = z * z
    z4 = z2 * z2
    z8 = z4 * z4
    kern = z + z2 + z4 + z8 + z8 * z8
    acc = jnp.sum(kern * (s_r * s))

    @pl.when(jnp.logical_and(v == 0, t == 0))
    def _():
        dl_ref[...] = jnp.zeros((1, 1), jnp.float32)

    dl_ref[...] = dl_ref[...] + (-acc / (N_SEL * N_SEL)).reshape(1, 1)


def _mmd(Ts, srows, scols):
    return pl.pallas_call(
        _mmd_body,
        grid=(2, 2 * B // _MMD_R),
        in_specs=[
            pl.BlockSpec((1, 2 * B, C), lambda v, t: (v, 0, 0)),
            pl.BlockSpec((1, _MMD_R, C), lambda v, t: (v, t, 0)),
            pl.BlockSpec((1, 1, 2 * B), lambda v, t: (v, 0, 0)),
            pl.BlockSpec((1, _MMD_R, 1), lambda v, t: (v, t, 0)),
        ],
        out_specs=pl.BlockSpec((1, 1), lambda v, t: (0, 0)),
        out_shape=jax.ShapeDtypeStruct((1, 1), jnp.float32),
        interpret=_INTERPRET,
    )(Ts, Ts, srows, scols)


def _dec_body(f_ref, w1_ref, b1_ref, w2_ref, b2_ref, w3_ref, b3_ref,
              w4_ref, b4_ref, o_ref):
    h = _lrelu(_dotT16(f_ref[...], w1_ref[...]) + b1_ref[...])
    h = _lrelu(_dotT16(h, w2_ref[...]) + b2_ref[...])
    h = _lrelu(_dotT16(h, w3_ref[...]) + b3_ref[...])
    o_ref[...] = _dotT16(h, w4_ref[...]) + b4_ref[...]


def _decoder(fused, w1, b1, w2, b2, w3, b3, w4, b4):
    din = w4.shape[0]
    return pl.pallas_call(
        _dec_body,
        out_shape=jax.ShapeDtypeStruct((B, din), jnp.float32),
        interpret=_INTERPRET,
    )(fused, w1, b1.reshape(1, -1), w2, b2.reshape(1, -1),
      w3, b3.reshape(1, -1), w4, b4.reshape(1, -1))


def kernel(x0, x1, noise0, noise1, W_pre0, b_pre0, W_pre1, b_pre1, W_router,
           enc_w1, enc_b1, enc_w2, enc_b2, enc_w3, enc_b3, enc_w4, enc_b4,
           dec0_w1, dec0_b1, dec0_w2, dec0_b2, dec0_w3, dec0_b3, dec0_w4,
           dec0_b4, dec1_w1, dec1_b1, dec1_w2, dec1_b2, dec1_w3, dec1_b3,
           dec1_w4, dec1_b4):
    m0 = _pre(x0, W_pre0, b_pre0)
    m1 = _pre(x1, W_pre1, b_pre1)

    ei0, ohg1_0, ohg2_0, bal0 = _route(m0, noise0, W_router)
    ei1, ohg1_1, ohg2_1, bal1 = _route(m1, noise1, W_router)

    ei = jnp.stack([ei0.reshape(K, E, F), ei1.reshape(K, E, F)], axis=0)
    ei = jnp.transpose(ei, (2, 0, 1, 3)).reshape(E, 2 * K, F)

    eo = _encoder(ei, enc_w1, enc_b1, enc_w2, enc_b2, enc_w3, enc_b3,
                  enc_w4, enc_b4)

    mid0_0, mid1_0, o0 = _combine(eo[:, 0, :], eo[:, 1, :], ohg1_0, ohg2_0)
    mid0_1, mid1_1, o1 = _combine(eo[:, 2, :], eo[:, 3, :], ohg1_1, ohg2_1)

    srows = []
    for seed in (0, 1):
        w0, w1 = _mmd_masks(seed)
        srows.append(np.concatenate([w0, -w1]).reshape(1, 2 * B))
    srows_np = np.stack(srows)
    srows = jnp.asarray(srows_np)
    scols = jnp.asarray(np.transpose(srows_np, (0, 2, 1)))

    Ts = jnp.stack([jnp.concatenate([mid0_0, mid1_0], axis=0),
                    jnp.concatenate([mid0_1, mid1_1], axis=0)])
    dist = _mmd(Ts, srows, scols)

    fused = jnp.concatenate([o0, o1], axis=1)
    rec0 = _decoder(fused, dec0_w1, dec0_b1, dec0_w2, dec0_b2,
                    dec0_w3, dec0_b3, dec0_w4, dec0_b4)
    rec1 = _decoder(fused, dec1_w1, dec1_b1, dec1_w2, dec1_b2,
                    dec1_w3, dec1_b3, dec1_w4, dec1_b4)

    bal = (bal0 + bal1).reshape(())
    dl = dist.reshape(())
    return fused, rec0, rec1, bal, dl


# speedup vs baseline: 1.3275x; 1.0484x over previous
import numpy as np
import jax
import jax.numpy as jnp
from jax.experimental import pallas as pl
from jax.experimental.pallas import tpu as pltpu

B = 1024
E = 8
K = 2
F = 512
C = 128
N_SEL = 920
N_TOT = 2 * N_SEL

_INTERPRET = False


def _dotT(a, w):
    return jax.lax.dot_general(a, w, (((1,), (1,)), ((), ())),
                               preferred_element_type=jnp.float32)


def _dotT16(a, w):
    return jax.lax.dot_general(a.astype(jnp.bfloat16), w.astype(jnp.bfloat16),
                               (((1,), (1,)), ((), ())),
                               preferred_element_type=jnp.float32)


def _lrelu(x):
    return jnp.where(x >= 0, x, 0.01 * x)


def _preroute_body(x_ref, w_ref, b_ref, noise_ref, wr_ref,
                   ei_ref, ohg1_ref, ohg2_ref, bal_ref):
    m = _dotT16(x_ref[...], w_ref[...]) + b_ref[...]
    sel = _dotT(m, wr_ref[...]) + noise_ref[...]
    eidx = jax.lax.broadcasted_iota(jnp.int32, (B, E), 1)
    g1 = jnp.max(sel, axis=1, keepdims=True)
    i1 = jnp.argmax(sel, axis=1)[:, None]
    oh1 = (eidx == i1).astype(jnp.float32)
    sel2 = jnp.where(oh1 > 0, -jnp.inf, sel)
    g2 = jnp.max(sel2, axis=1, keepdims=True)
    i2 = jnp.argmax(sel2, axis=1)[:, None]
    oh2 = (eidx == i2).astype(jnp.float32)
    ohg1_ref[...] = oh1 * g1
    ohg2_ref[...] = oh2 * g2
    d1 = oh1 * (g1 != 0).astype(jnp.float32)
    d2 = oh2 * (g2 != 0).astype(jnp.float32)
    ohd = jnp.concatenate([d1, d2], axis=1)
    ei_ref[...] = jax.lax.dot_general(ohd, m, (((0,), (0,)), ((), ())),
                                      preferred_element_type=jnp.float32)
    colsum = jnp.sum(0.5 * (oh1 + oh2), axis=0, keepdims=True)
    proxy = jnp.mean(sel, axis=0, keepdims=True)
    bal_ref[...] = (jnp.sum(proxy * colsum) * (E * E) / (B * E)).reshape(1, 1)


def _preroute(x, w, b, noise, wr):
    return pl.pallas_call(
        _preroute_body,
        out_shape=(
            jax.ShapeDtypeStruct((2 * E, F), jnp.float32),
            jax.ShapeDtypeStruct((B, E), jnp.float32),
            jax.ShapeDtypeStruct((B, E), jnp.float32),
            jax.ShapeDtypeStruct((1, 1), jnp.float32),
        ),
        interpret=_INTERPRET,
    )(x, w, b.reshape(1, F), noise, wr)


def _enc_body(ei_ref, w1_ref, b1_ref, w2_ref, b2_ref, w3_ref, b3_ref,
              w4_ref, b4_ref, eo_ref):
    x = ei_ref[0]
    h = jnp.maximum(_dotT(x, w1_ref[0]) + b1_ref[0], 0.0)
    h = jnp.maximum(_dotT(h, w2_ref[0]) + b2_ref[0], 0.0)
    h = jnp.maximum(_dotT(h, w3_ref[0]) + b3_ref[0], 0.0)
    eo_ref[0] = _dotT(h, w4_ref[0]) + b4_ref[0]


def _encoder(ei, w1, b1, w2, b2, w3, b3, w4, b4):
    n4 = 2 * K
    spec_w = lambda s: pl.BlockSpec((1,) + s, lambda e: (e, 0, 0))
    return pl.pallas_call(
        _enc_body,
        grid=(E,),
        in_specs=[
            pl.BlockSpec((1, n4, F), lambda e: (e, 0, 0)),
            spec_w((500, F)), spec_w((1, 500)),
            spec_w((500, 500)), spec_w((1, 500)),
            spec_w((2000, 500)), spec_w((1, 2000)),
            spec_w((C, 2000)), spec_w((1, C)),
        ],
        out_specs=pl.BlockSpec((1, n4, C), lambda e: (e, 0, 0)),
        out_shape=jax.ShapeDtypeStruct((E, n4, C), jnp.float32),
        interpret=_INTERPRET,
    )(ei, w1, b1.reshape(E, 1, 500), w2, b2.reshape(E, 1, 500),
      w3, b3.reshape(E, 1, 2000), w4, b4.reshape(E, 1, C))


def _combine_body(eo_ref, ohg1_0_ref, ohg2_0_ref, ohg1_1_ref, ohg2_1_ref,
                  Ts_ref, fused_ref):
    eo = eo_ref[...]
    m00 = jnp.dot(ohg1_0_ref[...], eo[:, 0, :],
                  preferred_element_type=jnp.float32)
    m10 = jnp.dot(ohg2_0_ref[...], eo[:, 1, :],
                  preferred_element_type=jnp.float32)
    m01 = jnp.dot(ohg1_1_ref[...], eo[:, 2, :],
                  preferred_element_type=jnp.float32)
    m11 = jnp.dot(ohg2_1_ref[...], eo[:, 3, :],
                  preferred_element_type=jnp.float32)
    Ts_ref[0, :B, :] = m00
    Ts_ref[0, B:, :] = m10
    Ts_ref[1, :B, :] = m01
    Ts_ref[1, B:, :] = m11
    fused_ref[:, :C] = m00 + m10
    fused_ref[:, C:] = m01 + m11


def _combine(eo, ohg1_0, ohg2_0, ohg1_1, ohg2_1):
    return pl.pallas_call(
        _combine_body,
        out_shape=(
            jax.ShapeDtypeStruct((2, 2 * B, C), jnp.float32),
            jax.ShapeDtypeStruct((B, 2 * C), jnp.float32),
        ),
        interpret=_INTERPRET,
    )(eo, ohg1_0, ohg2_0, ohg1_1, ohg2_1)


def _mmd_masks(seed):
    rng = np.random.default_rng(seed)
    i1 = rng.permutation(B)[:N_SEL]
    i2 = rng.permutation(B)[:N_SEL]
    w0 = np.zeros((B,), np.float32)
    w0[i1] = 1.0
    w1 = np.zeros((B,), np.float32)
    w1[i2] = 1.0
    return w0, w1


_MMD_R = 512
_PAIR_ROW = (0, 0, 0, 0, 1, 1, 1, 2, 2, 3)
_PAIR_COL = (0, 1, 2, 3, 1, 2, 3, 2, 3, 3)
_N_PAIR = len(_PAIR_ROW)


def _mmd_body(T_ref, Ta_ref, Tb_ref, srow_ref, sb_ref, sa_ref, wgt_ref,
              dl_ref):
    v = pl.program_id(0)
    p = pl.program_id(1)
    T = T_ref[0]
    s = srow_ref[0]
    m = jnp.abs(s)
    sq = jnp.sum(T * T, axis=1, keepdims=True)
    S1 = jnp.sum(jnp.dot(m, sq, preferred_element_type=jnp.float32))
    sumvec = jnp.dot(m, T, preferred_element_type=jnp.float32)
    sum_l2 = 2.0 * N_TOT * S1 - 2.0 * jnp.sum(sumvec * sumvec)
    bw = sum_l2 / (N_TOT * N_TOT - N_TOT) / 4.0
    Ta = Ta_ref[0]
    Tb = Tb_ref[0]
    sq_a = jnp.sum(Ta * Ta, axis=1, keepdims=True)
    sq_b = jnp.sum(Tb * Tb, axis=1, keepdims=True)
    s_a = sa_ref[0]
    s_b = sb_ref[0]
    G = jax.lax.dot_general(Ta, Tb, (((1,), (1,)), ((), ())),
                            preferred_element_type=jnp.float32)
    L2 = sq_a + jnp.transpose(sq_b) - 2.0 * G
    z = jnp.exp(-L2 / (16.0 * bw))
    z2 = z * z
    z4 = z2 * z2
    z8 = z4 * z4
    kern = z + z2 + z4 + z8 + z8 * z8
    acc = jnp.sum(kern * (s_a * s_b)) * wgt_ref[0, 0, 0]

    @pl.when(jnp.logical_and(v == 0, p == 0))
    def _():
        dl_ref[...] = jnp.zeros((1, 1), jnp.float32)

    dl_ref[...] = dl_ref[...] + (-acc / (N_SEL * N_SEL)).reshape(1, 1)


def _mmd(Ts, srows, scols):
    row = _PAIR_ROW
    col = _PAIR_COL
    wgts = jnp.asarray(
        np.array([1.0 if r == c else 2.0 for r, c in zip(row, col)],
                 np.float32).reshape(_N_PAIR, 1, 1))
    rmap = lambda v, p: (v, sum(jnp.where(p == i, r, 0) for i, r in enumerate(row)), 0)
    cmap = lambda v, p: (v, sum(jnp.where(p == i, c, 0) for i, c in enumerate(col)), 0)
    return pl.pallas_call(
        _mmd_body,
        grid=(2, _N_PAIR),
        in_specs=[
            pl.BlockSpec((1, 2 * B, C), lambda v, p: (v, 0, 0)),
            pl.BlockSpec((1, _MMD_R, C), rmap),
            pl.BlockSpec((1, _MMD_R, C), cmap),
            pl.BlockSpec((1, 1, 2 * B), lambda v, p: (v, 0, 0)),
            pl.BlockSpec((1, 1, _MMD_R),
                         lambda v, p: (v, 0, cmap(v, p)[1])),
            pl.BlockSpec((1, _MMD_R, 1),
                         lambda v, p: (v, rmap(v, p)[1], 0)),
            pl.BlockSpec((1, 1, 1), lambda v, p: (p, 0, 0)),
        ],
        out_specs=pl.BlockSpec((1, 1), lambda v, p: (0, 0)),
        out_shape=jax.ShapeDtypeStruct((1, 1), jnp.float32),
        interpret=_INTERPRET,
    )(Ts, Ts, Ts, srows, srows, scols, wgts)


def _dec_body(f_ref, w1_ref, b1_ref, w2_ref, b2_ref, w3_ref, b3_ref,
              w4_ref, b4_ref, o_ref):
    h = _lrelu(_dotT16(f_ref[...], w1_ref[...]) + b1_ref[...])
    h = _lrelu(_dotT16(h, w2_ref[...]) + b2_ref[...])
    h = _lrelu(_dotT16(h, w3_ref[...]) + b3_ref[...])
    o_ref[...] = _dotT16(h, w4_ref[...]) + b4_ref[...]


def _decoder(fused, w1, b1, w2, b2, w3, b3, w4, b4):
    din = w4.shape[0]
    return pl.pallas_call(
        _dec_body,
        out_shape=jax.ShapeDtypeStruct((B, din), jnp.float32),
        interpret=_INTERPRET,
    )(fused, w1, b1.reshape(1, -1), w2, b2.reshape(1, -1),
      w3, b3.reshape(1, -1), w4, b4.reshape(1, -1))


def kernel(x0, x1, noise0, noise1, W_pre0, b_pre0, W_pre1, b_pre1, W_router,
           enc_w1, enc_b1, enc_w2, enc_b2, enc_w3, enc_b3, enc_w4, enc_b4,
           dec0_w1, dec0_b1, dec0_w2, dec0_b2, dec0_w3, dec0_b3, dec0_w4,
           dec0_b4, dec1_w1, dec1_b1, dec1_w2, dec1_b2, dec1_w3, dec1_b3,
           dec1_w4, dec1_b4):
    ei0, ohg1_0, ohg2_0, bal0 = _preroute(x0, W_pre0, b_pre0, noise0, W_router)
    ei1, ohg1_1, ohg2_1, bal1 = _preroute(x1, W_pre1, b_pre1, noise1, W_router)

    ei = jnp.stack([ei0.reshape(K, E, F), ei1.reshape(K, E, F)], axis=0)
    ei = jnp.transpose(ei, (2, 0, 1, 3)).reshape(E, 2 * K, F)

    eo = _encoder(ei, enc_w1, enc_b1, enc_w2, enc_b2, enc_w3, enc_b3,
                  enc_w4, enc_b4)

    Ts, fused = _combine(eo, ohg1_0, ohg2_0, ohg1_1, ohg2_1)

    srows = []
    for seed in (0, 1):
        w0, w1 = _mmd_masks(seed)
        srows.append(np.concatenate([w0, -w1]).reshape(1, 2 * B))
    srows_np = np.stack(srows)
    srows = jnp.asarray(srows_np)
    scols = jnp.asarray(np.transpose(srows_np, (0, 2, 1)))

    dist = _mmd(Ts, srows, scols)

    rec0 = _decoder(fused, dec0_w1, dec0_b1, dec0_w2, dec0_b2,
                    dec0_w3, dec0_b3, dec0_w4, dec0_b4)
    rec1 = _decoder(fused, dec1_w1, dec1_b1, dec1_w2, dec1_b2,
                    dec1_w3, dec1_b3, dec1_w4, dec1_b4)

    bal = (bal0 + bal1).reshape(())
    dl = dist.reshape(())
    return fused, rec0, rec1, bal, dl


# speedup vs baseline: 1.3931x; 1.0494x over previous
import numpy as np
import jax
import jax.numpy as jnp
from jax.experimental import pallas as pl
from jax.experimental.pallas import tpu as pltpu

B = 1024
E = 8
K = 2
F = 512
C = 128
N_SEL = 920
N_TOT = 2 * N_SEL

_INTERPRET = False


def _dotT(a, w):
    return jax.lax.dot_general(a, w, (((1,), (1,)), ((), ())),
                               preferred_element_type=jnp.float32)


def _dotT16(a, w):
    return jax.lax.dot_general(a.astype(jnp.bfloat16), w.astype(jnp.bfloat16),
                               (((1,), (1,)), ((), ())),
                               preferred_element_type=jnp.float32)


def _lrelu(x):
    return jnp.where(x >= 0, x, 0.01 * x)


def _route_one(x, w, b, noise, wr):
    m = _dotT16(x, w) + b
    sel = _dotT(m, wr) + noise
    eidx = jax.lax.broadcasted_iota(jnp.int32, (B, E), 1)
    g1 = jnp.max(sel, axis=1, keepdims=True)
    i1 = jnp.argmax(sel, axis=1)[:, None]
    oh1 = (eidx == i1).astype(jnp.float32)
    sel2 = jnp.where(oh1 > 0, -jnp.inf, sel)
    g2 = jnp.max(sel2, axis=1, keepdims=True)
    i2 = jnp.argmax(sel2, axis=1)[:, None]
    oh2 = (eidx == i2).astype(jnp.float32)
    d1 = oh1 * (g1 != 0).astype(jnp.float32)
    d2 = oh2 * (g2 != 0).astype(jnp.float32)
    ohd = jnp.stack([d1, d2], axis=2).reshape(B, 2 * E)
    ei = jax.lax.dot_general(ohd, m, (((0,), (0,)), ((), ())),
                             preferred_element_type=jnp.float32)
    colsum = jnp.sum(0.5 * (oh1 + oh2), axis=0, keepdims=True)
    proxy = jnp.mean(sel, axis=0, keepdims=True)
    bal = jnp.sum(proxy * colsum) * (E * E) / (B * E)
    return ei.reshape(E, K, F), oh1 * g1, oh2 * g2, bal


def _preroute_body(x0_ref, w0_ref, b0_ref, n0_ref, x1_ref, w1_ref, b1_ref,
                   n1_ref, wr_ref, ei_ref, g10_ref, g20_ref, g11_ref,
                   g21_ref, bal_ref):
    wr = wr_ref[...]
    ei0, g10, g20, bal0 = _route_one(x0_ref[...], w0_ref[...], b0_ref[...],
                                     n0_ref[...], wr)
    ei1, g11, g21, bal1 = _route_one(x1_ref[...], w1_ref[...], b1_ref[...],
                                     n1_ref[...], wr)
    ei_ref[...] = jnp.concatenate([ei0, ei1], axis=1)
    g10_ref[...] = g10
    g20_ref[...] = g20
    g11_ref[...] = g11
    g21_ref[...] = g21
    bal_ref[...] = (bal0 + bal1).reshape(1, 1)


def _preroute(x0, w0, b0, n0, x1, w1, b1, n1, wr):
    return pl.pallas_call(
        _preroute_body,
        out_shape=(
            jax.ShapeDtypeStruct((E, 2 * K, F), jnp.float32),
            jax.ShapeDtypeStruct((B, E), jnp.float32),
            jax.ShapeDtypeStruct((B, E), jnp.float32),
            jax.ShapeDtypeStruct((B, E), jnp.float32),
            jax.ShapeDtypeStruct((B, E), jnp.float32),
            jax.ShapeDtypeStruct((1, 1), jnp.float32),
        ),
        interpret=_INTERPRET,
    )(x0, w0, b0.reshape(1, F), n0, x1, w1, b1.reshape(1, F), n1, wr)


def _enc_body(ei_ref, w1_ref, b1_ref, w2_ref, b2_ref, w3_ref, b3_ref,
              w4_ref, b4_ref, eo_ref):
    x = ei_ref[0]
    h = jnp.maximum(_dotT(x, w1_ref[0]) + b1_ref[0], 0.0)
    h = jnp.maximum(_dotT(h, w2_ref[0]) + b2_ref[0], 0.0)
    h = jnp.maximum(_dotT(h, w3_ref[0]) + b3_ref[0], 0.0)
    eo_ref[0] = _dotT(h, w4_ref[0]) + b4_ref[0]


def _encoder(ei, w1, b1, w2, b2, w3, b3, w4, b4):
    n4 = 2 * K
    spec_w = lambda s: pl.BlockSpec((1,) + s, lambda e: (e, 0, 0))
    return pl.pallas_call(
        _enc_body,
        grid=(E,),
        in_specs=[
            pl.BlockSpec((1, n4, F), lambda e: (e, 0, 0)),
            spec_w((500, F)), spec_w((1, 500)),
            spec_w((500, 500)), spec_w((1, 500)),
            spec_w((2000, 500)), spec_w((1, 2000)),
            spec_w((C, 2000)), spec_w((1, C)),
        ],
        out_specs=pl.BlockSpec((1, n4, C), lambda e: (e, 0, 0)),
        out_shape=jax.ShapeDtypeStruct((E, n4, C), jnp.float32),
        interpret=_INTERPRET,
    )(ei, w1, b1.reshape(E, 1, 500), w2, b2.reshape(E, 1, 500),
      w3, b3.reshape(E, 1, 2000), w4, b4.reshape(E, 1, C))


def _mmd_masks(seed):
    rng = np.random.default_rng(seed)
    i1 = rng.permutation(B)[:N_SEL]
    i2 = rng.permutation(B)[:N_SEL]
    w0 = np.zeros((B,), np.float32)
    w0[i1] = 1.0
    w1 = np.zeros((B,), np.float32)
    w1[i2] = 1.0
    return w0, w1


_MMD_R = 512
_PAIR_ROW = (0, 0, 0, 0, 1, 1, 1, 2, 2, 3)
_PAIR_COL = (0, 1, 2, 3, 1, 2, 3, 2, 3, 3)
_N_PAIR = len(_PAIR_ROW)
_N_STEP = 1 + 2 * _N_PAIR


def _mmd_tables():
    off_r, off_c, wgt = [], [], []
    for v in range(2):
        for r, c in zip(_PAIR_ROW, _PAIR_COL):
            off_r.append(v * 2 * B // _MMD_R + r)
            off_c.append(v * 2 * B // _MMD_R + c)
            wgt.append(1.0 if r == c else 2.0)
    return (np.asarray(off_r, np.int32), np.asarray(off_c, np.int32),
            np.asarray(wgt, np.float32),
            np.asarray([v for v in (0,) * _N_PAIR + (1,) * _N_PAIR],
                       np.int32))


def _cmmd_body(eo_ref, g10_ref, g20_ref, g11_ref, g21_ref, srow_ref,
               scol_ref, offr_ref, offc_ref, wgt_ref, vv_ref,
               fused_ref, dl_ref, Ts_scr, bw_scr):
    p = pl.program_id(0)

    @pl.when(p == 0)
    def _():
        eo = eo_ref[...]
        m00 = jnp.dot(g10_ref[...], eo[:, 0, :],
                      preferred_element_type=jnp.float32)
        m10 = jnp.dot(g20_ref[...], eo[:, 1, :],
                      preferred_element_type=jnp.float32)
        m01 = jnp.dot(g11_ref[...], eo[:, 2, :],
                      preferred_element_type=jnp.float32)
        m11 = jnp.dot(g21_ref[...], eo[:, 3, :],
                      preferred_element_type=jnp.float32)
        Ts_scr[0 * B:1 * B, :] = m00
        Ts_scr[1 * B:2 * B, :] = m10
        Ts_scr[2 * B:3 * B, :] = m01
        Ts_scr[3 * B:4 * B, :] = m11
        fused_ref[:, :C] = m00 + m10
        fused_ref[:, C:] = m01 + m11
        for v in range(2):
            T = Ts_scr[v * 2 * B:(v + 1) * 2 * B, :]
            mrow = jnp.abs(srow_ref[0, :, v * 2 * B:(v + 1) * 2 * B])
            sq = jnp.sum(T * T, axis=1, keepdims=True)
            S1 = jnp.sum(jnp.dot(mrow, sq,
                                 preferred_element_type=jnp.float32))
            sv = jnp.dot(mrow, T, preferred_element_type=jnp.float32)
            sum_l2 = 2.0 * N_TOT * S1 - 2.0 * jnp.sum(sv * sv)
            bw_scr[v] = sum_l2 / (N_TOT * N_TOT - N_TOT) / 4.0
        dl_ref[...] = jnp.zeros((1, 1), jnp.float32)

    @pl.when(p > 0)
    def _():
        i = jnp.maximum(p - 1, 0)
        orow = offr_ref[i] * _MMD_R
        ocol = offc_ref[i] * _MMD_R
        w = wgt_ref[i]
        vv = vv_ref[i]
        bw = bw_scr[vv]
        Ta = Ts_scr[pl.ds(orow, _MMD_R), :]
        Tb = Ts_scr[pl.ds(ocol, _MMD_R), :]
        sq_a = jnp.sum(Ta * Ta, axis=1, keepdims=True)
        sq_b = jnp.sum(Tb * Tb, axis=1, keepdims=True)
        s_a = scol_ref[pl.ds(orow, _MMD_R), :]
        s_b = srow_ref[0, :, pl.ds(ocol, _MMD_R)]
        G = jax.lax.dot_general(Ta, Tb, (((1,), (1,)), ((), ())),
                                preferred_element_type=jnp.float32)
        L2 = sq_a + jnp.transpose(sq_b) - 2.0 * G
        z = jnp.exp(-L2 / (16.0 * bw))
        z2 = z * z
        z4 = z2 * z2
        z8 = z4 * z4
        kern = z + z2 + z4 + z8 + z8 * z8
        acc = jnp.sum(kern * (s_a * s_b)) * w
        dl_ref[...] = dl_ref[...] + (-acc / (N_SEL * N_SEL)).reshape(1, 1)


def _combine_mmd(eo, g10, g20, g11, g21, srow, scol, offr, offc, wgt, vv):
    smem = lambda: pl.BlockSpec(memory_space=pltpu.SMEM)
    return pl.pallas_call(
        _cmmd_body,
        grid=(_N_STEP,),
        in_specs=[
            pl.BlockSpec((E, 2 * K, C), lambda p: (0, 0, 0)),
            pl.BlockSpec((B, E), lambda p: (0, 0)),
            pl.BlockSpec((B, E), lambda p: (0, 0)),
            pl.BlockSpec((B, E), lambda p: (0, 0)),
            pl.BlockSpec((B, E), lambda p: (0, 0)),
            pl.BlockSpec((1, 1, 4 * B), lambda p: (0, 0, 0)),
            pl.BlockSpec((4 * B, 1), lambda p: (0, 0)),
            smem(), smem(), smem(), smem(),
        ],
        out_specs=(
            pl.BlockSpec((B, 2 * C), lambda p: (0, 0)),
            pl.BlockSpec((1, 1), lambda p: (0, 0)),
        ),
        out_shape=(
            jax.ShapeDtypeStruct((B, 2 * C), jnp.float32),
            jax.ShapeDtypeStruct((1, 1), jnp.float32),
        ),
        scratch_shapes=[
            pltpu.VMEM((4 * B, C), jnp.float32),
            pltpu.SMEM((2,), jnp.float32),
        ],
        interpret=_INTERPRET,
    )(eo, g10, g20, g11, g21, srow, scol, offr, offc, wgt, vv)


def _dec_chain(f, w1, b1, w2, b2, w3, b3, w4, b4):
    h = _lrelu(_dotT16(f, w1) + b1)
    h = _lrelu(_dotT16(h, w2) + b2)
    h = _lrelu(_dotT16(h, w3) + b3)
    return _dotT16(h, w4) + b4


def _dec_body(f_ref, aw1, ab1, aw2, ab2, aw3, ab3, aw4, ab4,
              bw1, bb1, bw2, bb2, bw3, bb3, bw4, bb4, o0_ref, o1_ref):
    f = f_ref[...]
    o0_ref[...] = _dec_chain(f, aw1[...], ab1[...], aw2[...], ab2[...],
                             aw3[...], ab3[...], aw4[...], ab4[...])
    o1_ref[...] = _dec_chain(f, bw1[...], bb1[...], bw2[...], bb2[...],
                             bw3[...], bb3[...], bw4[...], bb4[...])


def _decoders(fused, p0, p1):
    return pl.pallas_call(
        _dec_body,
        out_shape=(
            jax.ShapeDtypeStruct((B, p0[6].shape[0]), jnp.float32),
            jax.ShapeDtypeStruct((B, p1[6].shape[0]), jnp.float32),
        ),
        interpret=_INTERPRET,
    )(fused, p0[0], p0[1].reshape(1, -1), p0[2], p0[3].reshape(1, -1),
      p0[4], p0[5].reshape(1, -1), p0[6], p0[7].reshape(1, -1),
      p1[0], p1[1].reshape(1, -1), p1[2], p1[3].reshape(1, -1),
      p1[4], p1[5].reshape(1, -1), p1[6], p1[7].reshape(1, -1))


def kernel(x0, x1, noise0, noise1, W_pre0, b_pre0, W_pre1, b_pre1, W_router,
           enc_w1, enc_b1, enc_w2, enc_b2, enc_w3, enc_b3, enc_w4, enc_b4,
           dec0_w1, dec0_b1, dec0_w2, dec0_b2, dec0_w3, dec0_b3, dec0_w4,
           dec0_b4, dec1_w1, dec1_b1, dec1_w2, dec1_b2, dec1_w3, dec1_b3,
           dec1_w4, dec1_b4):
    ei, g10, g20, g11, g21, bal = _preroute(
        x0, W_pre0, b_pre0, noise0, x1, W_pre1, b_pre1, noise1, W_router)

    eo = _encoder(ei, enc_w1, enc_b1, enc_w2, enc_b2, enc_w3, enc_b3,
                  enc_w4, enc_b4)

    srows = []
    for seed in (0, 1):
        w0m, w1m = _mmd_masks(seed)
        srows.append(np.concatenate([w0m, -w1m]))
    srow_np = np.concatenate(srows).reshape(1, 1, 4 * B)
    offr, offc, wgt, vv = _mmd_tables()

    fused, dist = _combine_mmd(
        eo, g10, g20, g11, g21, jnp.asarray(srow_np),
        jnp.asarray(srow_np.reshape(4 * B, 1)), jnp.asarray(offr),
        jnp.asarray(offc), jnp.asarray(wgt), jnp.asarray(vv))

    rec0, rec1 = _decoders(
        fused,
        (dec0_w1, dec0_b1, dec0_w2, dec0_b2, dec0_w3, dec0_b3, dec0_w4,
         dec0_b4),
        (dec1_w1, dec1_b1, dec1_w2, dec1_b2, dec1_w3, dec1_b3, dec1_w4,
         dec1_b4))

    return fused, rec0, rec1, bal.reshape(()), dist.reshape(())


# speedup vs baseline: 1.5920x; 1.1428x over previous
import numpy as np
import jax
import jax.numpy as jnp
from jax.experimental import pallas as pl
from jax.experimental.pallas import tpu as pltpu

B = 1024
E = 8
K = 2
F = 512
C = 128
N_SEL = 920
N_TOT = 2 * N_SEL

_INTERPRET = False


def _dotT(a, w):
    return jax.lax.dot_general(a, w, (((1,), (1,)), ((), ())),
                               preferred_element_type=jnp.float32)


def _dotT16(a, w):
    return jax.lax.dot_general(a.astype(jnp.bfloat16), w.astype(jnp.bfloat16),
                               (((1,), (1,)), ((), ())),
                               preferred_element_type=jnp.float32)


def _lrelu(x):
    return jnp.where(x >= 0, x, 0.01 * x)


def _route_one(x, w, b, noise, wr):
    m = _dotT16(x, w) + b
    sel = _dotT(m, wr) + noise
    eidx = jax.lax.broadcasted_iota(jnp.int32, (B, E), 1)
    g1 = jnp.max(sel, axis=1, keepdims=True)
    i1 = jnp.argmax(sel, axis=1)[:, None]
    oh1 = (eidx == i1).astype(jnp.float32)
    sel2 = jnp.where(oh1 > 0, -jnp.inf, sel)
    g2 = jnp.max(sel2, axis=1, keepdims=True)
    i2 = jnp.argmax(sel2, axis=1)[:, None]
    oh2 = (eidx == i2).astype(jnp.float32)
    d1 = oh1 * (g1 != 0).astype(jnp.float32)
    d2 = oh2 * (g2 != 0).astype(jnp.float32)
    ohd = jnp.concatenate([d1, d2], axis=1)
    ei = jax.lax.dot_general(ohd, m, (((0,), (0,)), ((), ())),
                             preferred_element_type=jnp.float32)
    colsum = jnp.sum(0.5 * (oh1 + oh2), axis=0, keepdims=True)
    proxy = jnp.mean(sel, axis=0, keepdims=True)
    bal = jnp.sum(proxy * colsum) * (E * E) / (B * E)
    return ei, oh1 * g1, oh2 * g2, bal


def _preroute_body(x0_ref, w0_ref, b0_ref, n0_ref, x1_ref, w1_ref, b1_ref,
                   n1_ref, wr_ref, ei_ref, g10_ref, g20_ref, g11_ref,
                   g21_ref, bal_ref):
    wr = wr_ref[...]
    ei0, g10, g20, bal0 = _route_one(x0_ref[...], w0_ref[...], b0_ref[...],
                                     n0_ref[...], wr)
    ei1, g11, g21, bal1 = _route_one(x1_ref[...], w1_ref[...], b1_ref[...],
                                     n1_ref[...], wr)
    ei_ref[:, 0:1, :] = ei0[0:E, :][:, None, :]
    ei_ref[:, 1:2, :] = ei0[E:2 * E, :][:, None, :]
    ei_ref[:, 2:3, :] = ei1[0:E, :][:, None, :]
    ei_ref[:, 3:4, :] = ei1[E:2 * E, :][:, None, :]
    g10_ref[...] = g10
    g20_ref[...] = g20
    g11_ref[...] = g11
    g21_ref[...] = g21
    bal_ref[...] = (bal0 + bal1).reshape(1, 1)


def _preroute(x0, w0, b0, n0, x1, w1, b1, n1, wr):
    return pl.pallas_call(
        _preroute_body,
        out_shape=(
            jax.ShapeDtypeStruct((E, 2 * K, F), jnp.float32),
            jax.ShapeDtypeStruct((B, E), jnp.float32),
            jax.ShapeDtypeStruct((B, E), jnp.float32),
            jax.ShapeDtypeStruct((B, E), jnp.float32),
            jax.ShapeDtypeStruct((B, E), jnp.float32),
            jax.ShapeDtypeStruct((1, 1), jnp.float32),
        ),
        interpret=_INTERPRET,
    )(x0, w0, b0.reshape(1, F), n0, x1, w1, b1.reshape(1, F), n1, wr)


def _enc_body(ei_ref, w1_ref, b1_ref, w2_ref, b2_ref, w3_ref, b3_ref,
              w4_ref, b4_ref, eo_ref):
    x = ei_ref[0]
    h = jnp.maximum(_dotT(x, w1_ref[0]) + b1_ref[0], 0.0)
    h = jnp.maximum(_dotT(h, w2_ref[0]) + b2_ref[0], 0.0)
    h = jnp.maximum(_dotT(h, w3_ref[0]) + b3_ref[0], 0.0)
    eo_ref[0] = _dotT(h, w4_ref[0]) + b4_ref[0]


def _encoder(ei, w1, b1, w2, b2, w3, b3, w4, b4):
    n4 = 2 * K
    spec_w = lambda s: pl.BlockSpec((1,) + s, lambda e: (e, 0, 0))
    return pl.pallas_call(
        _enc_body,
        grid=(E,),
        in_specs=[
            pl.BlockSpec((1, n4, F), lambda e: (e, 0, 0)),
            spec_w((500, F)), spec_w((1, 500)),
            spec_w((500, 500)), spec_w((1, 500)),
            spec_w((2000, 500)), spec_w((1, 2000)),
            spec_w((C, 2000)), spec_w((1, C)),
        ],
        out_specs=pl.BlockSpec((1, n4, C), lambda e: (e, 0, 0)),
        out_shape=jax.ShapeDtypeStruct((E, n4, C), jnp.float32),
        interpret=_INTERPRET,
    )(ei, w1, b1.reshape(E, 1, 500), w2, b2.reshape(E, 1, 500),
      w3, b3.reshape(E, 1, 2000), w4, b4.reshape(E, 1, C))


def _mmd_masks(seed):
    rng = np.random.default_rng(seed)
    i1 = rng.permutation(B)[:N_SEL]
    i2 = rng.permutation(B)[:N_SEL]
    w0 = np.zeros((B,), np.float32)
    w0[i1] = 1.0
    w1 = np.zeros((B,), np.float32)
    w1[i2] = 1.0
    return w0, w1


_MMD_R = 512
_PAIR_ROW = (0, 0, 0, 0, 1, 1, 1, 2, 2, 3)
_PAIR_COL = (0, 1, 2, 3, 1, 2, 3, 2, 3, 3)
_N_PAIR = len(_PAIR_ROW)
_N_STEP = 1 + 2 * _N_PAIR


def _mmd_tables():
    off_r, off_c, wgt = [], [], []
    for v in range(2):
        for r, c in zip(_PAIR_ROW, _PAIR_COL):
            off_r.append(v * 2 * B // _MMD_R + r)
            off_c.append(v * 2 * B // _MMD_R + c)
            wgt.append(1.0 if r == c else 2.0)
    return (np.asarray(off_r, np.int32), np.asarray(off_c, np.int32),
            np.asarray(wgt, np.float32),
            np.asarray([v for v in (0,) * _N_PAIR + (1,) * _N_PAIR],
                       np.int32))


_DEC_W_SHAPES = ((2000, 256), (500, 2000), (500, 500), (2048, 500),
                 (2000, 256), (500, 2000), (500, 500), (1024, 500))


def _cmmd_body(eo_ref, g10_ref, g20_ref, g11_ref, g21_ref, srow_ref,
               scol_ref, offr_ref, offc_ref, wgt_ref, vv_ref,
               w0_hbm, w1_hbm, w2_hbm, w3_hbm, w4_hbm, w5_hbm, w6_hbm,
               w7_hbm, ab1, ab2, ab3, ab4, bb1, bb2, bb3, bb4,
               fused_ref, dl_ref, o0_ref, o1_ref,
               Ts_scr, bw_scr, ws0, ws1, ws2, ws3, ws4, ws5, ws6, ws7,
               dsem):
    p = pl.program_id(0)
    whbm = (w0_hbm, w1_hbm, w2_hbm, w3_hbm, w4_hbm, w5_hbm, w6_hbm, w7_hbm)
    wscr = (ws0, ws1, ws2, ws3, ws4, ws5, ws6, ws7)

    @pl.when(p == 0)
    def _():
        for i in range(8):
            pltpu.make_async_copy(whbm[i], wscr[i], dsem.at[i]).start()
        eo = eo_ref[...]
        m00 = jnp.dot(g10_ref[...], eo[:, 0, :],
                      preferred_element_type=jnp.float32)
        m10 = jnp.dot(g20_ref[...], eo[:, 1, :],
                      preferred_element_type=jnp.float32)
        m01 = jnp.dot(g11_ref[...], eo[:, 2, :],
                      preferred_element_type=jnp.float32)
        m11 = jnp.dot(g21_ref[...], eo[:, 3, :],
                      preferred_element_type=jnp.float32)
        Ts_scr[0 * B:1 * B, :] = m00
        Ts_scr[1 * B:2 * B, :] = m10
        Ts_scr[2 * B:3 * B, :] = m01
        Ts_scr[3 * B:4 * B, :] = m11
        fused_ref[:, :C] = m00 + m10
        fused_ref[:, C:] = m01 + m11
        for v in range(2):
            T = Ts_scr[v * 2 * B:(v + 1) * 2 * B, :]
            mrow = jnp.abs(srow_ref[0, :, v * 2 * B:(v + 1) * 2 * B])
            sq = jnp.sum(T * T, axis=1, keepdims=True)
            S1 = jnp.sum(jnp.dot(mrow, sq,
                                 preferred_element_type=jnp.float32))
            sv = jnp.dot(mrow, T, preferred_element_type=jnp.float32)
            sum_l2 = 2.0 * N_TOT * S1 - 2.0 * jnp.sum(sv * sv)
            bw_scr[v] = sum_l2 / (N_TOT * N_TOT - N_TOT) / 4.0
        dl_ref[...] = jnp.zeros((1, 1), jnp.float32)

    @pl.when(jnp.logical_and(p > 0, p < _N_STEP))
    def _():
        i = jnp.minimum(jnp.maximum(p - 1, 0), 2 * _N_PAIR - 1)
        orow = offr_ref[i] * _MMD_R
        ocol = offc_ref[i] * _MMD_R
        w = wgt_ref[i]
        vv = vv_ref[i]
        bw = bw_scr[vv]
        Ta = Ts_scr[pl.ds(orow, _MMD_R), :]
        Tb = Ts_scr[pl.ds(ocol, _MMD_R), :]
        sq_a = jnp.sum(Ta * Ta, axis=1, keepdims=True)
        sq_b = jnp.sum(Tb * Tb, axis=1, keepdims=True)
        s_a = scol_ref[pl.ds(orow, _MMD_R), :]
        s_b = srow_ref[0, :, pl.ds(ocol, _MMD_R)]
        G = jax.lax.dot_general(Ta, Tb, (((1,), (1,)), ((), ())),
                                preferred_element_type=jnp.float32)
        L2 = sq_a + jnp.transpose(sq_b) - 2.0 * G
        z = jnp.exp(-L2 / (16.0 * bw))
        z2 = z * z
        z4 = z2 * z2
        z8 = z4 * z4
        kern = z + z2 + z4 + z8 + z8 * z8
        acc = jnp.sum(kern * (s_a * s_b)) * w
        dl_ref[...] = dl_ref[...] + (-acc / (N_SEL * N_SEL)).reshape(1, 1)

    @pl.when(p == _N_STEP)
    def _():
        for i in range(8):
            pltpu.make_async_copy(whbm[i], wscr[i], dsem.at[i]).wait()
        f = fused_ref[...]
        o0_ref[...] = _dec_chain(f, ws0[...], ab1[...], ws1[...], ab2[...],
                                 ws2[...], ab3[...], ws3[...], ab4[...])
        o1_ref[...] = _dec_chain(f, ws4[...], bb1[...], ws5[...], bb2[...],
                                 ws6[...], bb3[...], ws7[...], bb4[...])


def _combine_mmd(eo, g10, g20, g11, g21, srow, scol, offr, offc, wgt, vv,
                 p0, p1):
    smem = lambda: pl.BlockSpec(memory_space=pltpu.SMEM)
    anyspec = lambda: pl.BlockSpec(memory_space=pl.ANY)
    bias = lambda n: pl.BlockSpec((1, n), lambda p: (0, 0))
    return pl.pallas_call(
        _cmmd_body,
        grid=(_N_STEP + 1,),
        in_specs=[
            pl.BlockSpec((E, 2 * K, C), lambda p: (0, 0, 0)),
            pl.BlockSpec((B, E), lambda p: (0, 0)),
            pl.BlockSpec((B, E), lambda p: (0, 0)),
            pl.BlockSpec((B, E), lambda p: (0, 0)),
            pl.BlockSpec((B, E), lambda p: (0, 0)),
            pl.BlockSpec((1, 1, 4 * B), lambda p: (0, 0, 0)),
            pl.BlockSpec((4 * B, 1), lambda p: (0, 0)),
            smem(), smem(), smem(), smem(),
            anyspec(), anyspec(), anyspec(), anyspec(),
            anyspec(), anyspec(), anyspec(), anyspec(),
            bias(2000), bias(500), bias(500), bias(2048),
            bias(2000), bias(500), bias(500), bias(1024),
        ],
        out_specs=(
            pl.BlockSpec((B, 2 * C), lambda p: (0, 0)),
            pl.BlockSpec((1, 1), lambda p: (0, 0)),
            pl.BlockSpec((B, 2048), lambda p: (0, 0)),
            pl.BlockSpec((B, 1024), lambda p: (0, 0)),
        ),
        out_shape=(
            jax.ShapeDtypeStruct((B, 2 * C), jnp.float32),
            jax.ShapeDtypeStruct((1, 1), jnp.float32),
            jax.ShapeDtypeStruct((B, 2048), jnp.float32),
            jax.ShapeDtypeStruct((B, 1024), jnp.float32),
        ),
        scratch_shapes=[
            pltpu.VMEM((4 * B, C), jnp.float32),
            pltpu.SMEM((2,), jnp.float32),
        ] + [pltpu.VMEM(s, jnp.float32) for s in _DEC_W_SHAPES]
          + [pltpu.SemaphoreType.DMA((8,))],
        interpret=_INTERPRET,
    )(eo, g10, g20, g11, g21, srow, scol, offr, offc, wgt, vv,
      p0[0], p0[2], p0[4], p0[6], p1[0], p1[2], p1[4], p1[6],
      p0[1].reshape(1, -1), p0[3].reshape(1, -1), p0[5].reshape(1, -1),
      p0[7].reshape(1, -1), p1[1].reshape(1, -1), p1[3].reshape(1, -1),
      p1[5].reshape(1, -1), p1[7].reshape(1, -1))


def _dec_chain(f, w1, b1, w2, b2, w3, b3, w4, b4):
    h = _lrelu(_dotT16(f, w1) + b1)
    h = _lrelu(_dotT16(h, w2) + b2)
    h = _lrelu(_dotT16(h, w3) + b3)
    return _dotT16(h, w4) + b4


def kernel(x0, x1, noise0, noise1, W_pre0, b_pre0, W_pre1, b_pre1, W_router,
           enc_w1, enc_b1, enc_w2, enc_b2, enc_w3, enc_b3, enc_w4, enc_b4,
           dec0_w1, dec0_b1, dec0_w2, dec0_b2, dec0_w3, dec0_b3, dec0_w4,
           dec0_b4, dec1_w1, dec1_b1, dec1_w2, dec1_b2, dec1_w3, dec1_b3,
           dec1_w4, dec1_b4):
    ei, g10, g20, g11, g21, bal = _preroute(
        x0, W_pre0, b_pre0, noise0, x1, W_pre1, b_pre1, noise1, W_router)

    eo = _encoder(ei, enc_w1, enc_b1, enc_w2, enc_b2, enc_w3, enc_b3,
                  enc_w4, enc_b4)

    srows = []
    for seed in (0, 1):
        w0m, w1m = _mmd_masks(seed)
        srows.append(np.concatenate([w0m, -w1m]))
    srow_np = np.concatenate(srows).reshape(1, 1, 4 * B)
    offr, offc, wgt, vv = _mmd_tables()

    fused, dist, rec0, rec1 = _combine_mmd(
        eo, g10, g20, g11, g21, jnp.asarray(srow_np),
        jnp.asarray(srow_np.reshape(4 * B, 1)), jnp.asarray(offr),
        jnp.asarray(offc), jnp.asarray(wgt), jnp.asarray(vv),
        (dec0_w1, dec0_b1, dec0_w2, dec0_b2, dec0_w3, dec0_b3, dec0_w4,
         dec0_b4),
        (dec1_w1, dec1_b1, dec1_w2, dec1_b2, dec1_w3, dec1_b3, dec1_w4,
         dec1_b4))

    return fused, rec0, rec1, bal.reshape(()), dist.reshape(())
